# baseline (speedup 1.0000x reference)
import functools

import jax
import jax.numpy as jnp
from jax import lax
from jax.experimental import pallas as pl
from jax.experimental.pallas import tpu as pltpu
from jax.experimental.pallas import tpu_sc as plsc

N_NODES = 10000
N_EDGES = 320000
D = 128

NC = 2
NS = 16
NW = NC * NS
LANES = 16

NBLK = N_EDGES // D
BPW = 79
N_PAD = 10240
SL = N_PAD // NW
CH = 64
Q_SLOW = 192
Q_FAST = 2 * SL - Q_SLOW
NCH_MAX = Q_FAST // CH
ZROW = N_NODES

_mesh = plsc.VectorSubcoreMesh(core_axis_name="c", subcore_axis_name="s")
_sc_params = pltpu.CompilerParams(
    needs_layout_passes=False, use_tc_tiling_on_sc=False
)


@functools.partial(
    pl.kernel,
    mesh=_mesh,
    out_type=jax.ShapeDtypeStruct((NW, N_PAD), jnp.int32),
    compiler_params=_sc_params,
    scratch_types=[
        pltpu.VMEM((BPW, 2, D), jnp.int32),
        pltpu.VMEM((N_PAD,), jnp.int32),
    ],
)
def _lastsrc_kernel(ei_hbm, src_all, ei_v, src_v):
    wid = lax.axis_index("s") * NC + lax.axis_index("c")
    start = jnp.minimum(wid * (NBLK // NW) + jnp.minimum(wid, NBLK % NW),
                        NBLK - BPW)
    neg1 = jnp.full((LANES,), -1, jnp.int32)

    def init_body(j, carry):
        src_v[pl.ds(j * LANES, LANES)] = neg1
        return carry

    lax.fori_loop(0, N_PAD // LANES, init_body, 0)

    pltpu.sync_copy(ei_hbm.at[pl.ds(start, BPW)], ei_v)

    def blk_body(j, carry):
        for o in range(D // LANES):
            d16 = ei_v[j, 0, pl.ds(o * LANES, LANES)]
            s16 = ei_v[j, 1, pl.ds(o * LANES, LANES)]
            plsc.store_scatter(src_v, [d16], s16)
        return carry

    lax.fori_loop(0, BPW, blk_body, 0)
    pltpu.sync_copy(src_v, src_all.at[wid])


@functools.partial(
    pl.kernel,
    mesh=_mesh,
    out_type=jax.ShapeDtypeStruct((N_PAD, D), jnp.float32),
    compiler_params=_sc_params,
    scratch_types=[
        pltpu.VMEM((NW, Q_FAST), jnp.int32),
        pltpu.VMEM((Q_FAST,), jnp.int32),
        pltpu.VMEM((Q_FAST, D), jnp.float32),
        pltpu.SemaphoreType.DMA,
    ],
)
def _agg_kernel(src_all, xpad_hbm, agg_hbm, ssl_v, safe_v, rows_v, sem):
    c = lax.axis_index("c")
    s = lax.axis_index("s")
    base = s * (2 * SL) + c * Q_FAST
    rows = jnp.where(c == 0, Q_FAST, Q_SLOW)
    nch = rows // CH

    with jax.named_scope("ph_tbl"):
        pltpu.sync_copy(src_all.at[:, pl.ds(base, Q_FAST)], ssl_v)

    zrow = jnp.full((LANES,), ZROW, jnp.int32)

    def comb_body(j, carry):
        sl = pl.ds(j * LANES, LANES)
        bsrc = zrow
        for w in range(NW):
            sw = ssl_v[w, sl]
            bsrc = jnp.where(sw >= 0, sw, bsrc)
        safe_v[sl] = bsrc
        return carry

    with jax.named_scope("ph_comb"):
        lax.fori_loop(0, rows // LANES, comb_body, 0)

    with jax.named_scope("ph_fire"):
        for k in range(NCH_MAX):
            @pl.when(k < nch)
            def _fire(k=k):
                pltpu.async_copy(
                    xpad_hbm.at[safe_v.at[pl.ds(k * CH, CH)]],
                    rows_v.at[pl.ds(k * CH, CH)],
                    sem,
                )

    with jax.named_scope("ph_drain"):
        for k in range(NCH_MAX):
            @pl.when(k < nch)
            def _drain(k=k):
                pltpu.make_async_copy(
                    xpad_hbm.at[safe_v.at[pl.ds(k * CH, CH)]],
                    rows_v.at[pl.ds(k * CH, CH)],
                    sem,
                ).wait()

    with jax.named_scope("ph_write"):
        for k in range(NCH_MAX):
            @pl.when(k < nch)
            def _write(k=k):
                pltpu.sync_copy(
                    rows_v.at[pl.ds(k * CH, CH)],
                    agg_hbm.at[pl.ds(base + k * CH, CH)],
                )


def _mm1_body(x_ref, w_ref, b_ref, o_ref):
    dn = (((1,), (1,)), ((), ()))
    o_ref[...] = (
        lax.dot_general(x_ref[...], w_ref[:, :D], dn,
                        preferred_element_type=jnp.float32)
        + b_ref[...]
    )


def _mm2_body(y1_ref, agg_ref, w_ref, o_ref):
    dn = (((1,), (1,)), ((), ()))
    o_ref[...] = y1_ref[...] + lax.dot_general(
        agg_ref[...], w_ref[:, D:], dn, preferred_element_type=jnp.float32
    )


_ROWS_BLK = 2000
_GRID = (N_NODES // _ROWS_BLK,)
_ROW_SPEC = pl.BlockSpec((_ROWS_BLK, D), lambda i: (i, 0))
_W_SPEC = pl.BlockSpec((D, 2 * D), lambda i: (0, 0))
_OUT_TYPE = jax.ShapeDtypeStruct((N_NODES, D), jnp.float32)


def _mm1(x, W, b2d):
    return pl.pallas_call(
        _mm1_body,
        grid=_GRID,
        in_specs=[_ROW_SPEC, _W_SPEC, pl.BlockSpec((1, D), lambda i: (0, 0))],
        out_specs=_ROW_SPEC,
        out_shape=_OUT_TYPE,
    )(x, W, b2d)


def _mm2(y1, agg, W):
    return pl.pallas_call(
        _mm2_body,
        grid=_GRID,
        in_specs=[_ROW_SPEC, _ROW_SPEC, _W_SPEC],
        out_specs=_ROW_SPEC,
        out_shape=_OUT_TYPE,
    )(y1, agg, W)


@jax.jit
def kernel(x, edge_index, W, b):
    ei_t = jnp.transpose(edge_index.reshape(2, NBLK, D), (1, 0, 2))
    xpad = jnp.concatenate([x, jnp.zeros((LANES, D), x.dtype)], axis=0)
    src_all = _lastsrc_kernel(ei_t)
    agg = _agg_kernel(src_all, xpad)
    y1 = _mm1(x, W, b.reshape(1, D))
    return _mm2(y1, agg, W)


# speedup vs baseline: 1.2566x; 1.2566x over previous
import functools

import jax
import jax.numpy as jnp
from jax import lax
from jax.experimental import pallas as pl
from jax.experimental.pallas import tpu as pltpu
from jax.experimental.pallas import tpu_sc as plsc

N_NODES = 10000
N_EDGES = 320000
D = 128

NC = 2
NS = 16
NW = NC * NS
LANES = 16

NBLK = N_EDGES // D
BPW = 79
N_PAD = 10240
SL = N_PAD // NW
CH = 64
Q_SLOW = 320
Q_FAST = 2 * SL - Q_SLOW
NCH_MAX = Q_FAST // CH
ZROW = N_NODES
NZPAD = 256

_mesh = plsc.VectorSubcoreMesh(core_axis_name="c", subcore_axis_name="s")
_sc_params = pltpu.CompilerParams(
    needs_layout_passes=False, use_tc_tiling_on_sc=False
)


@functools.partial(
    pl.kernel,
    mesh=_mesh,
    out_type=jax.ShapeDtypeStruct((NW, N_PAD), jnp.int32),
    compiler_params=_sc_params,
    scratch_types=[
        pltpu.VMEM((BPW, 2, D), jnp.int32),
        pltpu.VMEM((N_PAD,), jnp.int32),
    ],
)
def _lastsrc_kernel(ei_hbm, src_all, ei_v, src_v):
    wid = lax.axis_index("s") * NC + lax.axis_index("c")
    start = jnp.minimum(wid * (NBLK // NW) + jnp.minimum(wid, NBLK % NW),
                        NBLK - BPW)
    neg1 = jnp.full((LANES,), -1, jnp.int32)

    def init_body(j, carry):
        src_v[pl.ds(j * LANES, LANES)] = neg1
        return carry

    lax.fori_loop(0, N_PAD // LANES, init_body, 0)

    pltpu.sync_copy(ei_hbm.at[pl.ds(start, BPW)], ei_v)

    def blk_body(j, carry):
        for o in range(D // LANES):
            d16 = ei_v[j, 0, pl.ds(o * LANES, LANES)]
            s16 = ei_v[j, 1, pl.ds(o * LANES, LANES)]
            plsc.store_scatter(src_v, [d16], s16)
        return carry

    lax.fori_loop(0, BPW, blk_body, 0)
    pltpu.sync_copy(src_v, src_all.at[wid])


@functools.partial(
    pl.kernel,
    mesh=_mesh,
    out_type=jax.ShapeDtypeStruct((N_PAD, D), jnp.float32),
    compiler_params=_sc_params,
    scratch_types=[
        pltpu.VMEM((NW, Q_FAST), jnp.int32),
        pltpu.VMEM((Q_FAST,), jnp.int32),
        pltpu.VMEM((Q_FAST, D), jnp.float32),
        pltpu.SemaphoreType.DMA,
    ],
)
def _agg_kernel(src_all, xpad_hbm, agg_hbm, ssl_v, safe_v, rows_v, sem):
    c = lax.axis_index("c")
    s = lax.axis_index("s")
    base = s * (2 * SL) + c * Q_FAST
    rows = jnp.where(c == 0, Q_FAST, Q_SLOW)
    nch = rows // CH

    with jax.named_scope("ph_tbl"):
        pltpu.sync_copy(src_all.at[:, pl.ds(base, Q_FAST)], ssl_v)

    iota = lax.iota(jnp.int32, LANES)

    def comb_body(j, carry):
        sl = pl.ds(j * LANES, LANES)
        bsrc = ZROW + ((base + j * LANES + iota) & (NZPAD - 1))
        for w in range(NW):
            sw = ssl_v[w, sl]
            bsrc = jnp.where(sw >= 0, sw, bsrc)
        safe_v[sl] = bsrc
        return carry

    with jax.named_scope("ph_comb"):
        lax.fori_loop(0, rows // LANES, comb_body, 0)

    with jax.named_scope("ph_fire"):
        for k in range(NCH_MAX):
            @pl.when(k < nch)
            def _fire(k=k):
                pltpu.async_copy(
                    xpad_hbm.at[safe_v.at[pl.ds(k * CH, CH)]],
                    rows_v.at[pl.ds(k * CH, CH)],
                    sem,
                )

    with jax.named_scope("ph_drain"):
        for k in range(NCH_MAX):
            @pl.when(k < nch)
            def _drain(k=k):
                pltpu.make_async_copy(
                    xpad_hbm.at[safe_v.at[pl.ds(k * CH, CH)]],
                    rows_v.at[pl.ds(k * CH, CH)],
                    sem,
                ).wait()

    with jax.named_scope("ph_write"):
        for k in range(NCH_MAX):
            @pl.when(k < nch)
            def _write(k=k):
                pltpu.sync_copy(
                    rows_v.at[pl.ds(k * CH, CH)],
                    agg_hbm.at[pl.ds(base + k * CH, CH)],
                )


def _mm1_body(x_ref, w_ref, b_ref, o_ref):
    dn = (((1,), (1,)), ((), ()))
    o_ref[...] = (
        lax.dot_general(x_ref[...], w_ref[:, :D], dn,
                        preferred_element_type=jnp.float32)
        + b_ref[...]
    )


def _mm2_body(y1_ref, agg_ref, w_ref, o_ref):
    dn = (((1,), (1,)), ((), ()))
    o_ref[...] = y1_ref[...] + lax.dot_general(
        agg_ref[...], w_ref[:, D:], dn, preferred_element_type=jnp.float32
    )


_ROWS_BLK = 2000
_GRID = (N_NODES // _ROWS_BLK,)
_ROW_SPEC = pl.BlockSpec((_ROWS_BLK, D), lambda i: (i, 0))
_W_SPEC = pl.BlockSpec((D, 2 * D), lambda i: (0, 0))
_OUT_TYPE = jax.ShapeDtypeStruct((N_NODES, D), jnp.float32)


def _mm1(x, W, b2d):
    return pl.pallas_call(
        _mm1_body,
        grid=_GRID,
        in_specs=[_ROW_SPEC, _W_SPEC, pl.BlockSpec((1, D), lambda i: (0, 0))],
        out_specs=_ROW_SPEC,
        out_shape=_OUT_TYPE,
    )(x, W, b2d)


def _mm2(y1, agg, W):
    return pl.pallas_call(
        _mm2_body,
        grid=_GRID,
        in_specs=[_ROW_SPEC, _ROW_SPEC, _W_SPEC],
        out_specs=_ROW_SPEC,
        out_shape=_OUT_TYPE,
    )(y1, agg, W)


@jax.jit
def kernel(x, edge_index, W, b):
    ei_t = jnp.transpose(edge_index.reshape(2, NBLK, D), (1, 0, 2))
    xpad = jnp.concatenate([x, jnp.zeros((NZPAD, D), x.dtype)], axis=0)
    src_all = _lastsrc_kernel(ei_t)
    agg = _agg_kernel(src_all, xpad)
    y1 = _mm1(x, W, b.reshape(1, D))
    return _mm2(y1, agg, W)


# speedup vs baseline: 1.3149x; 1.0464x over previous
import functools

import jax
import jax.numpy as jnp
from jax import lax
from jax.experimental import pallas as pl
from jax.experimental.pallas import tpu as pltpu
from jax.experimental.pallas import tpu_sc as plsc

N_NODES = 10000
N_EDGES = 320000
D = 128

NC = 2
NS = 16
NW = NC * NS
LANES = 16

NBLK = N_EDGES // D
BPW = 79
N_PAD = 10240
SL = N_PAD // NW
CH = 64
Q_SLOW = 320
Q_FAST = 2 * SL - Q_SLOW
NCH_MAX = Q_FAST // CH
ZROW = N_NODES
NZPAD = 256

_mesh = plsc.VectorSubcoreMesh(core_axis_name="c", subcore_axis_name="s")
_sc_params = pltpu.CompilerParams(
    needs_layout_passes=False, use_tc_tiling_on_sc=False
)


@functools.partial(
    pl.kernel,
    mesh=_mesh,
    out_type=jax.ShapeDtypeStruct((NW, N_PAD), jnp.int32),
    compiler_params=_sc_params,
    scratch_types=[
        pltpu.VMEM((BPW, 2, D), jnp.int32),
        pltpu.VMEM((N_PAD,), jnp.int32),
    ],
)
def _lastsrc_kernel(ei_hbm, src_all, ei_v, src_v):
    wid = lax.axis_index("s") * NC + lax.axis_index("c")
    start = jnp.minimum(wid * (NBLK // NW) + jnp.minimum(wid, NBLK % NW),
                        NBLK - BPW)
    neg1 = jnp.full((LANES,), -1, jnp.int32)

    def init_body(j, carry):
        for u in range(8):
            src_v[pl.ds((j * 8 + u) * LANES, LANES)] = neg1
        return carry

    lax.fori_loop(0, N_PAD // (8 * LANES), init_body, 0)

    pltpu.sync_copy(ei_hbm.at[pl.ds(start, BPW)], ei_v)

    def blk_body(j, carry):
        for o in range(D // LANES):
            d16 = ei_v[j, 0, pl.ds(o * LANES, LANES)]
            s16 = ei_v[j, 1, pl.ds(o * LANES, LANES)]
            plsc.store_scatter(src_v, [d16], s16)
        return carry

    lax.fori_loop(0, BPW, blk_body, 0)
    pltpu.sync_copy(src_v, src_all.at[wid])


@functools.partial(
    pl.kernel,
    mesh=_mesh,
    out_type=jax.ShapeDtypeStruct((N_PAD, D), jnp.float32),
    compiler_params=_sc_params,
    scratch_types=[
        pltpu.VMEM((NW, Q_FAST), jnp.int32),
        pltpu.VMEM((Q_FAST,), jnp.int32),
        pltpu.VMEM((Q_FAST, D), jnp.float32),
        pltpu.SemaphoreType.DMA,
    ],
)
def _agg_kernel(src_all, xpad_hbm, agg_hbm, ssl_v, safe_v, rows_v, sem):
    c = lax.axis_index("c")
    s = lax.axis_index("s")
    base = s * (2 * SL) + c * Q_FAST
    rows = jnp.where(c == 0, Q_FAST, Q_SLOW)
    nch = rows // CH

    pltpu.sync_copy(src_all.at[:, pl.ds(base, Q_FAST)], ssl_v)

    iota = lax.iota(jnp.int32, LANES)

    def comb_body(j, carry):
        sl = pl.ds(j * LANES, LANES)
        bsrc = ZROW + ((base + j * LANES + iota) & (NZPAD - 1))
        for w in range(NW):
            sw = ssl_v[w, sl]
            bsrc = jnp.where(sw >= 0, sw, bsrc)
        safe_v[sl] = bsrc
        return carry

    lax.fori_loop(0, rows // LANES, comb_body, 0)

    for k in range(NCH_MAX):
        @pl.when(k < nch)
        def _fire(k=k):
            pltpu.async_copy(
                xpad_hbm.at[safe_v.at[pl.ds(k * CH, CH)]],
                rows_v.at[pl.ds(k * CH, CH)],
                sem,
            )

    for k in range(NCH_MAX):
        @pl.when(k < nch)
        def _drain(k=k):
            pltpu.make_async_copy(
                xpad_hbm.at[safe_v.at[pl.ds(k * CH, CH)]],
                rows_v.at[pl.ds(k * CH, CH)],
                sem,
            ).wait()

    for k in range(NCH_MAX):
        @pl.when(k < nch)
        def _write(k=k):
            pltpu.sync_copy(
                rows_v.at[pl.ds(k * CH, CH)],
                agg_hbm.at[pl.ds(base + k * CH, CH)],
            )


def _mm1_body(x_ref, w_ref, b_ref, o_ref):
    dn = (((1,), (1,)), ((), ()))
    o_ref[...] = (
        lax.dot_general(x_ref[...], w_ref[:, :D], dn,
                        preferred_element_type=jnp.float32)
        + b_ref[...]
    )


def _mm2_body(y1_ref, agg_ref, w_ref, o_ref):
    dn = (((1,), (1,)), ((), ()))
    o_ref[...] = y1_ref[...] + lax.dot_general(
        agg_ref[...], w_ref[:, D:], dn, preferred_element_type=jnp.float32
    )


_ROWS_BLK = 2000
_GRID = (N_NODES // _ROWS_BLK,)
_ROW_SPEC = pl.BlockSpec((_ROWS_BLK, D), lambda i: (i, 0))
_W_SPEC = pl.BlockSpec((D, 2 * D), lambda i: (0, 0))
_OUT_TYPE = jax.ShapeDtypeStruct((N_NODES, D), jnp.float32)


def _mm1(x, W, b2d):
    return pl.pallas_call(
        _mm1_body,
        grid=_GRID,
        in_specs=[_ROW_SPEC, _W_SPEC, pl.BlockSpec((1, D), lambda i: (0, 0))],
        out_specs=_ROW_SPEC,
        out_shape=_OUT_TYPE,
    )(x, W, b2d)


def _mm2(y1, agg, W):
    return pl.pallas_call(
        _mm2_body,
        grid=_GRID,
        in_specs=[_ROW_SPEC, _ROW_SPEC, _W_SPEC],
        out_specs=_ROW_SPEC,
        out_shape=_OUT_TYPE,
    )(y1, agg, W)


@jax.jit
def kernel(x, edge_index, W, b):
    ei_t = jnp.transpose(edge_index.reshape(2, NBLK, D), (1, 0, 2))
    xpad = jnp.concatenate([x, jnp.zeros((NZPAD, D), x.dtype)], axis=0)
    src_all = _lastsrc_kernel(ei_t)
    agg = _agg_kernel(src_all, xpad)
    y1 = _mm1(x, W, b.reshape(1, D))
    return _mm2(y1, agg, W)


# speedup vs baseline: 1.3681x; 1.0405x over previous
import functools

import jax
import jax.numpy as jnp
from jax import lax
from jax.experimental import pallas as pl
from jax.experimental.pallas import tpu as pltpu
from jax.experimental.pallas import tpu_sc as plsc

N_NODES = 10000
N_EDGES = 320000
D = 128

NC = 2
NS = 16
NW = NC * NS
LANES = 16

NBLK = N_EDGES // D
BPW = 79
N_PAD = 10240
SL = N_PAD // NW
CH = 64
Q_SLOW = 320
Q_FAST = 2 * SL - Q_SLOW
NCH_MAX = Q_FAST // CH
ZROW = N_NODES
NZPAD = 256

_mesh = plsc.VectorSubcoreMesh(core_axis_name="c", subcore_axis_name="s")
_sc_params = pltpu.CompilerParams(
    needs_layout_passes=False, use_tc_tiling_on_sc=False
)


@functools.partial(
    pl.kernel,
    mesh=_mesh,
    out_type=jax.ShapeDtypeStruct((NW, N_PAD), jnp.int32),
    compiler_params=_sc_params,
    scratch_types=[
        pltpu.VMEM((BPW, 2, D), jnp.int32),
        pltpu.VMEM((N_PAD,), jnp.int32),
    ],
)
def _lastsrc_kernel(ei_hbm, src_all, ei_v, src_v):
    wid = lax.axis_index("s") * NC + lax.axis_index("c")
    start = jnp.minimum(wid * (NBLK // NW) + jnp.minimum(wid, NBLK % NW),
                        NBLK - BPW)
    neg1 = jnp.full((LANES,), -1, jnp.int32)

    def init_body(j, carry):
        for u in range(8):
            src_v[pl.ds((j * 8 + u) * LANES, LANES)] = neg1
        return carry

    lax.fori_loop(0, N_PAD // (8 * LANES), init_body, 0)

    pltpu.sync_copy(ei_hbm.at[pl.ds(start, BPW)], ei_v)

    def blk_body(j, carry):
        for o in range(D // LANES):
            d16 = ei_v[j, 0, pl.ds(o * LANES, LANES)]
            s16 = ei_v[j, 1, pl.ds(o * LANES, LANES)]
            plsc.store_scatter(src_v, [d16], s16)
        return carry

    lax.fori_loop(0, BPW, blk_body, 0)
    pltpu.sync_copy(src_v, src_all.at[wid])


@functools.partial(
    pl.kernel,
    mesh=_mesh,
    out_type=jax.ShapeDtypeStruct((N_PAD, D), jnp.float32),
    compiler_params=_sc_params,
    scratch_types=[
        pltpu.VMEM((NW, Q_FAST), jnp.int32),
        pltpu.VMEM((Q_FAST,), jnp.int32),
        pltpu.VMEM((Q_FAST, D), jnp.float32),
        pltpu.SemaphoreType.DMA,
    ],
)
def _agg_kernel(src_all, xpad_hbm, agg_hbm, ssl_v, safe_v, rows_v, sem):
    c = lax.axis_index("c")
    s = lax.axis_index("s")
    base = s * (2 * SL) + c * Q_FAST
    rows = jnp.where(c == 0, Q_FAST, Q_SLOW)
    nch = rows // CH

    pltpu.sync_copy(src_all.at[:, pl.ds(base, Q_FAST)], ssl_v)

    iota = lax.iota(jnp.int32, LANES)

    def comb_body(j, carry):
        sl = pl.ds(j * LANES, LANES)
        bsrc = ZROW + ((base + j * LANES + iota) & (NZPAD - 1))
        for w in range(NW):
            sw = ssl_v[w, sl]
            bsrc = jnp.where(sw >= 0, sw, bsrc)
        safe_v[sl] = bsrc
        return carry

    lax.fori_loop(0, rows // LANES, comb_body, 0)

    for k in range(NCH_MAX):
        @pl.when(k < nch)
        def _fire(k=k):
            pltpu.async_copy(
                xpad_hbm.at[safe_v.at[pl.ds(k * CH, CH)]],
                rows_v.at[pl.ds(k * CH, CH)],
                sem,
            )

    for k in range(NCH_MAX):
        @pl.when(k < nch)
        def _drain(k=k):
            pltpu.make_async_copy(
                xpad_hbm.at[safe_v.at[pl.ds(k * CH, CH)]],
                rows_v.at[pl.ds(k * CH, CH)],
                sem,
            ).wait()

    for k in range(NCH_MAX):
        @pl.when(k < nch)
        def _write(k=k):
            pltpu.sync_copy(
                rows_v.at[pl.ds(k * CH, CH)],
                agg_hbm.at[pl.ds(base + k * CH, CH)],
            )


def _mm1_body(x_ref, w_ref, b_ref, o_ref):
    dn = (((1,), (1,)), ((), ()))
    o_ref[...] = (
        lax.dot_general(x_ref[...], w_ref[:, :D], dn,
                        preferred_element_type=jnp.float32)
        + b_ref[...]
    )


def _mm2_body(y1_ref, agg_ref, w_ref, o_ref):
    dn = (((1,), (1,)), ((), ()))
    o_ref[...] = y1_ref[...] + lax.dot_general(
        agg_ref[...], w_ref[:, D:], dn, preferred_element_type=jnp.float32
    )


_ROWS_BLK = 5000
_GRID = (N_NODES // _ROWS_BLK,)
_ROW_SPEC = pl.BlockSpec((_ROWS_BLK, D), lambda i: (i, 0))
_W_SPEC = pl.BlockSpec((D, 2 * D), lambda i: (0, 0))
_OUT_TYPE = jax.ShapeDtypeStruct((N_NODES, D), jnp.float32)


def _mm1(x, W, b2d):
    return pl.pallas_call(
        _mm1_body,
        grid=_GRID,
        in_specs=[_ROW_SPEC, _W_SPEC, pl.BlockSpec((1, D), lambda i: (0, 0))],
        out_specs=_ROW_SPEC,
        out_shape=_OUT_TYPE,
    )(x, W, b2d)


def _mm2(y1, agg, W):
    return pl.pallas_call(
        _mm2_body,
        grid=_GRID,
        in_specs=[_ROW_SPEC, _ROW_SPEC, _W_SPEC],
        out_specs=_ROW_SPEC,
        out_shape=_OUT_TYPE,
    )(y1, agg, W)


@jax.jit
def kernel(x, edge_index, W, b):
    ei_t = jnp.transpose(edge_index.reshape(2, NBLK, D), (1, 0, 2))
    xpad = jnp.concatenate([x, jnp.zeros((NZPAD, D), x.dtype)], axis=0)
    src_all = _lastsrc_kernel(ei_t)
    agg = _agg_kernel(src_all, xpad)
    y1 = _mm1(x, W, b.reshape(1, D))
    return _mm2(y1, agg, W)


# speedup vs baseline: 1.3779x; 1.0071x over previous
import functools

import jax
import jax.numpy as jnp
from jax import lax
from jax.experimental import pallas as pl
from jax.experimental.pallas import tpu as pltpu
from jax.experimental.pallas import tpu_sc as plsc

N_NODES = 10000
N_EDGES = 320000
D = 128

NC = 2
NS = 16
NW = NC * NS
LANES = 16

NBLK = N_EDGES // D
BPW = 79
N_PAD = 10240
SL = N_PAD // NW
CH = 64
NCH = SL // CH
ZROW = N_NODES
NZPAD = 256

_mesh = plsc.VectorSubcoreMesh(core_axis_name="c", subcore_axis_name="s")
_sc_params = pltpu.CompilerParams(
    needs_layout_passes=False, use_tc_tiling_on_sc=False
)


@functools.partial(
    pl.kernel,
    mesh=_mesh,
    out_type=jax.ShapeDtypeStruct((NW, N_PAD), jnp.int32),
    compiler_params=_sc_params,
    scratch_types=[
        pltpu.VMEM((BPW, 2, D), jnp.int32),
        pltpu.VMEM((N_PAD,), jnp.int32),
    ],
)
def _lastsrc_kernel(ei_hbm, src_all, ei_v, src_v):
    wid = lax.axis_index("s") * NC + lax.axis_index("c")
    start = jnp.minimum(wid * (NBLK // NW) + jnp.minimum(wid, NBLK % NW),
                        NBLK - BPW)
    neg1 = jnp.full((LANES,), -1, jnp.int32)

    def init_body(j, carry):
        for u in range(8):
            src_v[pl.ds((j * 8 + u) * LANES, LANES)] = neg1
        return carry

    lax.fori_loop(0, N_PAD // (8 * LANES), init_body, 0)

    pltpu.sync_copy(ei_hbm.at[pl.ds(start, BPW)], ei_v)

    def blk_body(j, carry):
        for o in range(D // LANES):
            d16 = ei_v[j, 0, pl.ds(o * LANES, LANES)]
            s16 = ei_v[j, 1, pl.ds(o * LANES, LANES)]
            plsc.store_scatter(src_v, [d16], s16)
        return carry

    lax.fori_loop(0, BPW, blk_body, 0)
    pltpu.sync_copy(src_v, src_all.at[wid])


@functools.partial(
    pl.kernel,
    mesh=_mesh,
    out_type=jax.ShapeDtypeStruct((N_PAD, D), jnp.float32),
    compiler_params=_sc_params,
    scratch_types=[
        pltpu.VMEM((NW, SL), jnp.int32),
        pltpu.VMEM((SL,), jnp.int32),
        pltpu.VMEM((SL, D), jnp.float32),
        pltpu.SemaphoreType.DMA,
    ],
)
def _agg_kernel(src_all, xpad_hbm, agg_hbm, ssl_v, safe_v, rows_v, sem):
    wid = lax.axis_index("s") * NC + lax.axis_index("c")
    base = wid * SL

    pltpu.sync_copy(src_all.at[:, pl.ds(base, SL)], ssl_v)

    iota = lax.iota(jnp.int32, LANES)

    def comb_body(j, carry):
        sl = pl.ds(j * LANES, LANES)
        bsrc = ZROW + ((base + j * LANES + iota) & (NZPAD - 1))
        for w in range(NW):
            sw = ssl_v[w, sl]
            bsrc = jnp.where(sw >= 0, sw, bsrc)
        safe_v[sl] = bsrc
        return carry

    lax.fori_loop(0, SL // LANES, comb_body, 0)

    copies = [
        pltpu.async_copy(
            xpad_hbm.at[safe_v.at[pl.ds(k * CH, CH)]],
            rows_v.at[pl.ds(k * CH, CH)],
            sem,
        )
        for k in range(NCH)
    ]
    for cp in copies:
        cp.wait()
    pltpu.sync_copy(rows_v, agg_hbm.at[pl.ds(base, SL)])


def _mm1_body(x_ref, w_ref, b_ref, o_ref):
    dn = (((1,), (1,)), ((), ()))
    o_ref[...] = (
        lax.dot_general(x_ref[...], w_ref[:, :D], dn,
                        preferred_element_type=jnp.float32)
        + b_ref[...]
    )


def _mm2_body(y1_ref, agg_ref, w_ref, o_ref):
    dn = (((1,), (1,)), ((), ()))
    o_ref[...] = y1_ref[...] + lax.dot_general(
        agg_ref[...], w_ref[:, D:], dn, preferred_element_type=jnp.float32
    )


_ROWS_BLK = 5000
_GRID = (N_NODES // _ROWS_BLK,)
_ROW_SPEC = pl.BlockSpec((_ROWS_BLK, D), lambda i: (i, 0))
_W_SPEC = pl.BlockSpec((D, 2 * D), lambda i: (0, 0))
_OUT_TYPE = jax.ShapeDtypeStruct((N_NODES, D), jnp.float32)


def _mm1(x, W, b2d):
    return pl.pallas_call(
        _mm1_body,
        grid=_GRID,
        in_specs=[_ROW_SPEC, _W_SPEC, pl.BlockSpec((1, D), lambda i: (0, 0))],
        out_specs=_ROW_SPEC,
        out_shape=_OUT_TYPE,
    )(x, W, b2d)


def _mm2(y1, agg, W):
    return pl.pallas_call(
        _mm2_body,
        grid=_GRID,
        in_specs=[_ROW_SPEC, _ROW_SPEC, _W_SPEC],
        out_specs=_ROW_SPEC,
        out_shape=_OUT_TYPE,
    )(y1, agg, W)


@jax.jit
def kernel(x, edge_index, W, b):
    ei_t = jnp.transpose(edge_index.reshape(2, NBLK, D), (1, 0, 2))
    xpad = jnp.concatenate([x, jnp.zeros((NZPAD, D), x.dtype)], axis=0)
    src_all = _lastsrc_kernel(ei_t)
    agg = _agg_kernel(src_all, xpad)
    y1 = _mm1(x, W, b.reshape(1, D))
    return _mm2(y1, agg, W)


# speedup vs baseline: 1.3931x; 1.0110x over previous
import functools

import jax
import jax.numpy as jnp
from jax import lax
from jax.experimental import pallas as pl
from jax.experimental.pallas import tpu as pltpu
from jax.experimental.pallas import tpu_sc as plsc

N_NODES = 10000
N_EDGES = 320000
D = 128

NC = 2
NS = 16
NW = NC * NS
LANES = 16

NBLK = N_EDGES // D
BPW = 79
N_PAD = 10240
SL = N_PAD // NW
CH = 64
NCH = SL // CH
ZROW = N_NODES
NZPAD = 256

_mesh = plsc.VectorSubcoreMesh(core_axis_name="c", subcore_axis_name="s")
_sc_params = pltpu.CompilerParams(
    needs_layout_passes=False, use_tc_tiling_on_sc=False
)


@functools.partial(
    pl.kernel,
    mesh=_mesh,
    out_type=jax.ShapeDtypeStruct((NW, N_PAD), jnp.int32),
    compiler_params=_sc_params,
    scratch_types=[
        pltpu.VMEM((BPW, 2, D), jnp.int32),
        pltpu.VMEM((N_PAD,), jnp.int32),
        pltpu.SemaphoreType.DMA,
    ],
)
def _lastsrc_kernel(ei_hbm, src_all, ei_v, src_v, sem):
    wid = lax.axis_index("s") * NC + lax.axis_index("c")
    start = jnp.minimum(wid * (NBLK // NW) + jnp.minimum(wid, NBLK % NW),
                        NBLK - BPW)
    neg1 = jnp.full((LANES,), -1, jnp.int32)
    half = BPW // 2

    cp1 = pltpu.async_copy(
        ei_hbm.at[pl.ds(start, half)], ei_v.at[pl.ds(0, half)], sem
    )
    cp2 = pltpu.async_copy(
        ei_hbm.at[pl.ds(start + half, BPW - half)],
        ei_v.at[pl.ds(half, BPW - half)],
        sem,
    )

    def init_body(j, carry):
        for u in range(8):
            src_v[pl.ds((j * 8 + u) * LANES, LANES)] = neg1
        return carry

    lax.fori_loop(0, N_PAD // (8 * LANES), init_body, 0)

    def blk_body(j, carry):
        for o in range(D // LANES):
            d16 = ei_v[j, 0, pl.ds(o * LANES, LANES)]
            s16 = ei_v[j, 1, pl.ds(o * LANES, LANES)]
            plsc.store_scatter(src_v, [d16], s16)
        return carry

    cp1.wait()
    lax.fori_loop(0, half, blk_body, 0)
    cp2.wait()
    lax.fori_loop(half, BPW, blk_body, 0)
    pltpu.sync_copy(src_v, src_all.at[wid])


@functools.partial(
    pl.kernel,
    mesh=_mesh,
    out_type=jax.ShapeDtypeStruct((N_PAD, D), jnp.float32),
    compiler_params=_sc_params,
    scratch_types=[
        pltpu.VMEM((NW, SL), jnp.int32),
        pltpu.VMEM((SL,), jnp.int32),
        pltpu.VMEM((SL, D), jnp.float32),
        pltpu.SemaphoreType.DMA,
    ],
)
def _agg_kernel(src_all, xpad_hbm, agg_hbm, ssl_v, safe_v, rows_v, sem):
    wid = lax.axis_index("s") * NC + lax.axis_index("c")
    base = wid * SL

    pltpu.sync_copy(src_all.at[:, pl.ds(base, SL)], ssl_v)

    iota = lax.iota(jnp.int32, LANES)

    def comb_body(j, carry):
        sl = pl.ds(j * LANES, LANES)
        bsrc = ZROW + ((base + j * LANES + iota) & (NZPAD - 1))
        for w in range(NW):
            sw = ssl_v[w, sl]
            bsrc = jnp.where(sw >= 0, sw, bsrc)
        safe_v[sl] = bsrc
        return carry

    lax.fori_loop(0, SL // LANES, comb_body, 0)

    copies = [
        pltpu.async_copy(
            xpad_hbm.at[safe_v.at[pl.ds(k * CH, CH)]],
            rows_v.at[pl.ds(k * CH, CH)],
            sem,
        )
        for k in range(NCH)
    ]
    for cp in copies:
        cp.wait()
    pltpu.sync_copy(rows_v, agg_hbm.at[pl.ds(base, SL)])


def _mm1_body(x_ref, w_ref, b_ref, o_ref):
    dn = (((1,), (1,)), ((), ()))
    o_ref[...] = (
        lax.dot_general(x_ref[...], w_ref[:, :D], dn,
                        preferred_element_type=jnp.float32)
        + b_ref[...]
    )


def _mm2_body(y1_ref, agg_ref, w_ref, o_ref):
    dn = (((1,), (1,)), ((), ()))
    o_ref[...] = y1_ref[...] + lax.dot_general(
        agg_ref[...], w_ref[:, D:], dn, preferred_element_type=jnp.float32
    )


_ROWS_BLK = 5000
_GRID = (N_NODES // _ROWS_BLK,)
_ROW_SPEC = pl.BlockSpec((_ROWS_BLK, D), lambda i: (i, 0))
_W_SPEC = pl.BlockSpec((D, 2 * D), lambda i: (0, 0))
_OUT_TYPE = jax.ShapeDtypeStruct((N_NODES, D), jnp.float32)


def _mm1(x, W, b2d):
    return pl.pallas_call(
        _mm1_body,
        grid=_GRID,
        in_specs=[_ROW_SPEC, _W_SPEC, pl.BlockSpec((1, D), lambda i: (0, 0))],
        out_specs=_ROW_SPEC,
        out_shape=_OUT_TYPE,
    )(x, W, b2d)


def _mm2(y1, agg, W):
    return pl.pallas_call(
        _mm2_body,
        grid=_GRID,
        in_specs=[_ROW_SPEC, _ROW_SPEC, _W_SPEC],
        out_specs=_ROW_SPEC,
        out_shape=_OUT_TYPE,
    )(y1, agg, W)


@jax.jit
def kernel(x, edge_index, W, b):
    ei_t = jnp.transpose(edge_index.reshape(2, NBLK, D), (1, 0, 2))
    xpad = jnp.concatenate([x, jnp.zeros((NZPAD, D), x.dtype)], axis=0)
    src_all = _lastsrc_kernel(ei_t)
    agg = _agg_kernel(src_all, xpad)
    y1 = _mm1(x, W, b.reshape(1, D))
    return _mm2(y1, agg, W)
